# baseline (speedup 1.0000x reference)
import functools

import jax
import jax.numpy as jnp
from jax.experimental import pallas as pl
from jax.experimental.pallas import tpu as pltpu

N = 16384
H = 2048
D = 64


def _fused_kernel(nk, tar_ref, embs_ref, src_ref, o_ref, acc_ref):
    k = pl.program_id(0)

    @pl.when(k == 0)
    def _init():
        acc_ref[...] = jnp.zeros_like(acc_ref)

    @pl.when(k < nk)
    def _phase1():
        acc_ref[...] += jnp.dot(tar_ref[...], embs_ref[...],
                                preferred_element_type=jnp.float32)

    @pl.when(k >= nk)
    def _phase2():
        o_ref[...] = jnp.maximum(
            jnp.dot(src_ref[...], acc_ref[...],
                    preferred_element_type=jnp.float32),
            0.0)


@functools.partial(jax.jit, static_argnames=("tk", "tm"))
def _run(pois_embs, HG_poi_src, HG_poi_tar, tk=2048, tm=1024):
    nk = N // tk
    nm = N // tm
    return pl.pallas_call(
        functools.partial(_fused_kernel, nk),
        grid=(nk + nm,),
        in_specs=[
            pl.BlockSpec((H, tk), lambda k: (0, jnp.minimum(k, nk - 1))),
            pl.BlockSpec((tk, D), lambda k: (jnp.minimum(k, nk - 1), 0)),
            pl.BlockSpec((tm, H), lambda k: (jnp.maximum(k - nk, 0), 0)),
        ],
        out_specs=pl.BlockSpec((tm, D), lambda k: (jnp.maximum(k - nk, 0), 0)),
        out_shape=jax.ShapeDtypeStruct((N, D), jnp.float32),
        scratch_shapes=[pltpu.VMEM((H, D), jnp.float32)],
        compiler_params=pltpu.CompilerParams(
            dimension_semantics=("arbitrary",),
            vmem_limit_bytes=63 * 1024 * 1024),
    )(HG_poi_tar, pois_embs, HG_poi_src)


def kernel(pois_embs, HG_poi_src, HG_poi_tar):
    return _run(pois_embs, HG_poi_src, HG_poi_tar)


# speedup vs baseline: 1.0202x; 1.0202x over previous
import functools

import jax
import jax.numpy as jnp
from jax.experimental import pallas as pl
from jax.experimental.pallas import tpu as pltpu

N = 16384
H = 2048
D = 64


def _fused_kernel(nk, tar_ref, embs_ref, src_ref, o_ref, acc_ref):
    k = pl.program_id(0)

    @pl.when(k == 0)
    def _init():
        acc_ref[...] = jnp.zeros_like(acc_ref)

    @pl.when(k < nk)
    def _phase1():
        acc_ref[...] += jnp.dot(tar_ref[...], embs_ref[...],
                                preferred_element_type=jnp.float32)

    @pl.when(k >= nk)
    def _phase2():
        o_ref[...] = jnp.maximum(
            jnp.dot(src_ref[...], acc_ref[...],
                    preferred_element_type=jnp.float32),
            0.0)


@functools.partial(jax.jit, static_argnames=("tk", "tm"))
def _run(pois_embs, HG_poi_src, HG_poi_tar, tk=1024, tm=1024):
    nk = N // tk
    nm = N // tm
    return pl.pallas_call(
        functools.partial(_fused_kernel, nk),
        grid=(nk + nm,),
        in_specs=[
            pl.BlockSpec((H, tk), lambda k: (0, jnp.minimum(k, nk - 1))),
            pl.BlockSpec((tk, D), lambda k: (jnp.minimum(k, nk - 1), 0)),
            pl.BlockSpec((tm, H), lambda k: (jnp.maximum(k - nk, 0), 0)),
        ],
        out_specs=pl.BlockSpec((tm, D), lambda k: (jnp.maximum(k - nk, 0), 0)),
        out_shape=jax.ShapeDtypeStruct((N, D), jnp.float32),
        scratch_shapes=[pltpu.VMEM((H, D), jnp.float32)],
        compiler_params=pltpu.CompilerParams(
            dimension_semantics=("arbitrary",),
            vmem_limit_bytes=63 * 1024 * 1024),
    )(HG_poi_tar, pois_embs, HG_poi_src)


def kernel(pois_embs, HG_poi_src, HG_poi_tar):
    return _run(pois_embs, HG_poi_src, HG_poi_tar)
